# baseline (speedup 1.0000x reference)
import jax
import jax.numpy as jnp
from jax import lax
from jax.experimental import pallas as pl
from jax.experimental.pallas import tpu as pltpu
from jax.experimental.pallas import tpu_sc as plsc

N_NODES = 10000
D_FEAT = 128
N_EDGES = 320000

_NC = 2
_NS = 16
_NW = _NC * _NS
_EPT = N_EDGES // _NW
_K = 80
_NPASS = 5
_EPP = _EPT // _NPASS
_CPP = _EPP // _K
_NBUF = 4
_N_PAD = 10240
_RPT = _N_PAD // _NS
_OFF_TGT = _EPP
_OFF_VAL = 2 * _EPP


def _scale_chunk(rows, idx_p, ci):
    def gbody(g, _):
        vbits = idx_p[pl.ds(_OFF_VAL + ci * _K + g * 16, 16)]
        vv = plsc.bitcast(vbits, jnp.float32)
        blk = rows.at[pl.ds(g * 16, 16)]
        for j in range(16):
            vs = jnp.broadcast_to(vv[j], (16,))
            for d in range(D_FEAT // 16):
                sl = pl.ds(d * 16, 16)
                blk[j, sl] = blk[j, sl] * vs
        return 0

    lax.fori_loop(0, _K // 16, gbody, 0)


def _sc_body(x_hbm, packed_hbm, out_hbm,
             acc_sh, b0, b1, b2, b3, idx_p,
             g0, g1, g2, g3, s0, s1, s2, s3):
    bufs = (b0, b1, b2, b3)
    gsems = (g0, g1, g2, g3)
    ssems = (s0, s1, s2, s3)
    c = lax.axis_index("c")
    s = lax.axis_index("s")
    wid = s * _NC + c

    zeros16 = jnp.zeros((16,), jnp.float32)

    def zbody(i, _):
        for d in range(D_FEAT // 16):
            b0[i, pl.ds(d * 16, 16)] = zeros16
        return 0

    lax.fori_loop(0, _K, zbody, 0)
    r0 = s * _RPT
    for j in range(_RPT // _K):
        pltpu.sync_copy(b0, acc_sh.at[pl.ds(r0 + j * _K, _K)])
    plsc.subcore_barrier()

    def gather_start(ci, b):
        pltpu.async_copy(x_hbm.at[idx_p.at[pl.ds(ci * _K, _K)]],
                         bufs[b], gsems[b])

    def gather_wait(ci, b):
        pltpu.make_async_copy(x_hbm.at[idx_p.at[pl.ds(ci * _K, _K)]],
                              bufs[b], gsems[b]).wait()

    def scat_fire(ci, b):
        for g in range(_K // 16):
            tv = idx_p[pl.ds(_OFF_TGT + ci * _K + g * 16, 16)]
            pltpu.async_copy(bufs[b].at[pl.ds(g * 16, 16)], acc_sh.at[tv],
                             ssems[b], add=True)

    def scat_drain(ci, b):
        for g in range(_K // 16):
            tv = idx_p[pl.ds(_OFF_TGT + ci * _K + g * 16, 16)]
            pltpu.make_async_copy(bufs[b].at[pl.ds(g * 16, 16)],
                                  acc_sh.at[tv], ssems[b]).wait()

    for ps in range(_NPASS):
        p0 = (wid * _NPASS + ps) * 3 * _EPP
        pltpu.sync_copy(packed_hbm.at[pl.ds(p0, 3 * _EPP)], idx_p)

        for b in range(_NBUF - 1):
            gather_start(b, b)

        def step(q, b, guard_drain):
            gather_wait(q, b)
            pb = (b - 1) % _NBUF
            if guard_drain:
                @pl.when(q > 0)
                def _():
                    scat_drain(q - 1, pb)
            else:
                scat_drain(q - 1, pb)

            @pl.when(q + _NBUF - 1 < _CPP)
            def _():
                gather_start(q + _NBUF - 1, pb)

            _scale_chunk(bufs[b], idx_p, q)
            scat_fire(q, b)

        def pbody(p, _):
            for b in range(_NBUF):
                step(_NBUF * p + b, b, guard_drain=(b == 0))
            return 0

        ntail = _CPP % _NBUF
        nfull = _CPP // _NBUF
        lax.fori_loop(0, nfull, pbody, 0)
        for t in range(ntail):
            q = nfull * _NBUF + t
            step(q, q % _NBUF, guard_drain=False)
        scat_drain(_CPP - 1, (_CPP - 1) % _NBUF)

    plsc.subcore_barrier()

    for j in range(_RPT // _K):
        rj = r0 + j * _K
        pltpu.async_copy(acc_sh.at[pl.ds(rj, _K)],
                         out_hbm.at[c].at[pl.ds(rj, _K)], gsems[j % _NBUF])
    for j in range(_RPT // _K):
        rj = r0 + j * _K
        pltpu.make_async_copy(acc_sh.at[pl.ds(rj, _K)],
                              out_hbm.at[c].at[pl.ds(rj, _K)],
                              gsems[j % _NBUF]).wait()


def _tc_add_body(a_ref, b_ref, o_ref):
    o_ref[...] = a_ref[...] + b_ref[...]


def kernel(x_source, neighborhood_indices, neighborhood_values):
    tgt = neighborhood_indices[0]
    src = neighborhood_indices[1]
    packed = jnp.stack(
        [src.reshape(_NW, _NPASS, _EPP),
         tgt.reshape(_NW, _NPASS, _EPP),
         jax.lax.bitcast_convert_type(neighborhood_values,
                                      jnp.int32).reshape(_NW, _NPASS, _EPP)],
        axis=2).reshape(-1)

    mesh = plsc.VectorSubcoreMesh(core_axis_name="c", subcore_axis_name="s")
    partials = pl.kernel(
        _sc_body,
        mesh=mesh,
        compiler_params=pltpu.CompilerParams(needs_layout_passes=False),
        out_type=jax.ShapeDtypeStruct((_NC, _N_PAD, D_FEAT), jnp.float32),
        scratch_types=[
            pltpu.VMEM_SHARED((_N_PAD, D_FEAT), jnp.float32),
            pltpu.VMEM((_K, D_FEAT), jnp.float32),
            pltpu.VMEM((_K, D_FEAT), jnp.float32),
            pltpu.VMEM((_K, D_FEAT), jnp.float32),
            pltpu.VMEM((_K, D_FEAT), jnp.float32),
            pltpu.VMEM((3 * _EPP,), jnp.int32),
            pltpu.SemaphoreType.DMA,
            pltpu.SemaphoreType.DMA,
            pltpu.SemaphoreType.DMA,
            pltpu.SemaphoreType.DMA,
            pltpu.SemaphoreType.DMA,
            pltpu.SemaphoreType.DMA,
            pltpu.SemaphoreType.DMA,
            pltpu.SemaphoreType.DMA,
        ],
    )(x_source, packed)

    blk = 1000
    out = pl.pallas_call(
        _tc_add_body,
        out_shape=jax.ShapeDtypeStruct((N_NODES, D_FEAT), jnp.float32),
        grid=(N_NODES // blk,),
        in_specs=[
            pl.BlockSpec((blk, D_FEAT), lambda i: (i, 0)),
            pl.BlockSpec((blk, D_FEAT), lambda i: (i, 0)),
        ],
        out_specs=pl.BlockSpec((blk, D_FEAT), lambda i: (i, 0)),
    )(partials[0], partials[1])
    return out


# speedup vs baseline: 1.0600x; 1.0600x over previous
import jax
import jax.numpy as jnp
from jax import lax
from jax.experimental import pallas as pl
from jax.experimental.pallas import tpu as pltpu
from jax.experimental.pallas import tpu_sc as plsc

N_NODES = 10000
D_FEAT = 128
N_EDGES = 320000

_NC = 2
_NS = 16
_NW = _NC * _NS
_EPT = N_EDGES // _NW
_K = 80
_NPASS = 5
_EPP = _EPT // _NPASS
_CPP = _EPP // _K
_NBUF = 4
_N_PAD = 10240
_RPT = _N_PAD // _NS


def _scale_chunk(rows, vals_p, ci):
    def gbody(g, _):
        vv = vals_p[pl.ds(ci * _K + g * 16, 16)]
        blk = rows.at[pl.ds(g * 16, 16)]
        for j in range(16):
            vs = jnp.broadcast_to(vv[j], (16,))
            for d in range(D_FEAT // 16):
                sl = pl.ds(d * 16, 16)
                blk[j, sl] = blk[j, sl] * vs
        return 0

    lax.fori_loop(0, _K // 16, gbody, 0)


def _sc_body(x_hbm, src_hbm, tgt_hbm, vals_hbm, out_hbm,
             acc_sh, b0, b1, b2, b3, src_p, tgt_p, vals_p,
             g0, g1, g2, g3, s0, s1, s2, s3):
    bufs = (b0, b1, b2, b3)
    gsems = (g0, g1, g2, g3)
    ssems = (s0, s1, s2, s3)
    c = lax.axis_index("c")
    s = lax.axis_index("s")
    wid = s * _NC + c

    zeros16 = jnp.zeros((16,), jnp.float32)

    def zbody(i, _):
        for d in range(D_FEAT // 16):
            b0[i, pl.ds(d * 16, 16)] = zeros16
        return 0

    lax.fori_loop(0, _K, zbody, 0)
    r0 = s * _RPT
    for j in range(_RPT // _K):
        pltpu.sync_copy(b0, acc_sh.at[pl.ds(r0 + j * _K, _K)])
    plsc.subcore_barrier()

    def gather_start(ci, b):
        pltpu.async_copy(x_hbm.at[src_p.at[pl.ds(ci * _K, _K)]],
                         bufs[b], gsems[b])

    def gather_wait(ci, b):
        pltpu.make_async_copy(x_hbm.at[src_p.at[pl.ds(ci * _K, _K)]],
                              bufs[b], gsems[b]).wait()

    def scat_fire(ci, b):
        for g in range(_K // 16):
            tv = tgt_p[pl.ds(ci * _K + g * 16, 16)]
            pltpu.async_copy(bufs[b].at[pl.ds(g * 16, 16)], acc_sh.at[tv],
                             ssems[b], add=True)

    def scat_drain(ci, b):
        for g in range(_K // 16):
            tv = tgt_p[pl.ds(ci * _K + g * 16, 16)]
            pltpu.make_async_copy(bufs[b].at[pl.ds(g * 16, 16)],
                                  acc_sh.at[tv], ssems[b]).wait()

    for ps in range(_NPASS):
        e0 = wid * _EPT + ps * _EPP
        pltpu.sync_copy(src_hbm.at[pl.ds(e0, _EPP)], src_p)
        pltpu.sync_copy(tgt_hbm.at[pl.ds(e0, _EPP)], tgt_p)
        pltpu.sync_copy(vals_hbm.at[pl.ds(e0, _EPP)], vals_p)

        for b in range(_NBUF - 1):
            gather_start(b, b)

        def pbody(p, _):
            for b in range(_NBUF):
                q = _NBUF * p + b
                gather_wait(q, b)
                pb = (b - 1) % _NBUF
                if b == 0:
                    @pl.when(p > 0)
                    def _():
                        scat_drain(q - 1, pb)
                else:
                    scat_drain(q - 1, pb)

                @pl.when(q + _NBUF - 1 < _CPP)
                def _():
                    gather_start(q + _NBUF - 1, pb)

                _scale_chunk(bufs[b], vals_p, q)
                scat_fire(q, b)
            return 0

        ntail = _CPP % _NBUF
        nfull = _CPP // _NBUF
        lax.fori_loop(0, nfull, pbody, 0)
        for t in range(ntail):
            q = nfull * _NBUF + t
            b = q % _NBUF
            gather_wait(q, b)
            scat_drain(q - 1, (b - 1) % _NBUF)
            _scale_chunk(bufs[b], vals_p, q)
            scat_fire(q, b)
        scat_drain(_CPP - 1, (_CPP - 1) % _NBUF)

    plsc.subcore_barrier()

    for j in range(_RPT // _K):
        rr = r0 + j * _K
        pltpu.async_copy(acc_sh.at[pl.ds(rr, _K)],
                         out_hbm.at[c].at[pl.ds(rr, _K)], gsems[j % _NBUF])
    for j in range(_RPT // _K):
        rr = r0 + j * _K
        pltpu.make_async_copy(acc_sh.at[pl.ds(rr, _K)],
                              out_hbm.at[c].at[pl.ds(rr, _K)],
                              gsems[j % _NBUF]).wait()


def _tc_add_body(a_ref, b_ref, o_ref):
    o_ref[...] = a_ref[...] + b_ref[...]


def kernel(x_source, neighborhood_indices, neighborhood_values):
    tgt = neighborhood_indices[0]
    src = neighborhood_indices[1]

    mesh = plsc.VectorSubcoreMesh(core_axis_name="c", subcore_axis_name="s")
    partials = pl.kernel(
        _sc_body,
        mesh=mesh,
        compiler_params=pltpu.CompilerParams(needs_layout_passes=False),
        out_type=jax.ShapeDtypeStruct((_NC, _N_PAD, D_FEAT), jnp.float32),
        scratch_types=[
            pltpu.VMEM_SHARED((_N_PAD, D_FEAT), jnp.float32),
            pltpu.VMEM((_K, D_FEAT), jnp.float32),
            pltpu.VMEM((_K, D_FEAT), jnp.float32),
            pltpu.VMEM((_K, D_FEAT), jnp.float32),
            pltpu.VMEM((_K, D_FEAT), jnp.float32),
            pltpu.VMEM((_EPP,), jnp.int32),
            pltpu.VMEM((_EPP,), jnp.int32),
            pltpu.VMEM((_EPP,), jnp.float32),
            pltpu.SemaphoreType.DMA,
            pltpu.SemaphoreType.DMA,
            pltpu.SemaphoreType.DMA,
            pltpu.SemaphoreType.DMA,
            pltpu.SemaphoreType.DMA,
            pltpu.SemaphoreType.DMA,
            pltpu.SemaphoreType.DMA,
            pltpu.SemaphoreType.DMA,
        ],
    )(x_source, src, tgt, neighborhood_values)

    blk = 1000
    out = pl.pallas_call(
        _tc_add_body,
        out_shape=jax.ShapeDtypeStruct((N_NODES, D_FEAT), jnp.float32),
        grid=(N_NODES // blk,),
        in_specs=[
            pl.BlockSpec((blk, D_FEAT), lambda i: (i, 0)),
            pl.BlockSpec((blk, D_FEAT), lambda i: (i, 0)),
        ],
        out_specs=pl.BlockSpec((blk, D_FEAT), lambda i: (i, 0)),
    )(partials[0], partials[1])
    return out
